# baseline (speedup 1.0000x reference)
import functools

import jax
import jax.numpy as jnp
from jax import lax
from jax.experimental import pallas as pl
from jax.experimental.pallas import tpu as pltpu
from jax.experimental.pallas import tpu_sc as plsc

B = 4096
CTX = 20
EMBED = 64
NC = 2
NS = 16
NW = NC * NS
BPW = B // NW
NQ = 2
QB = BPW // NQ


def _sc_gather(i_t, o, w):
    mesh = plsc.VectorSubcoreMesh(core_axis_name="c", subcore_axis_name="s")

    @functools.partial(
        pl.kernel,
        mesh=mesh,
        compiler_params=pltpu.CompilerParams(use_tc_tiling_on_sc=False),
        out_type=(
            jax.ShapeDtypeStruct((B, EMBED), jnp.float32),
            jax.ShapeDtypeStruct((B, EMBED), jnp.float32),
        ),
        scratch_types=[
            pltpu.VMEM((CTX, BPW), jnp.int32),
            pltpu.VMEM((BPW,), jnp.int32),
            pltpu.VMEM((CTX, QB, EMBED), jnp.float32),
            pltpu.VMEM((BPW, EMBED), jnp.float32),
            pltpu.VMEM((BPW, EMBED), jnp.float32),
            pltpu.SemaphoreType.DMA,
        ],
    )
    def k(i_hbm, o_hbm, w_hbm, in_out, out_out,
          idx_v, oidx_v, bufs, acc_v, orow_v, sem):
        wid = lax.axis_index("s") * NC + lax.axis_index("c")
        base = wid * BPW

        pltpu.sync_copy(i_hbm.at[:, pl.ds(base, BPW)], idx_v)
        pltpu.sync_copy(o_hbm.at[pl.ds(base, BPW)], oidx_v)

        pltpu.async_copy(w_hbm.at[oidx_v], orow_v, sem).wait()
        pltpu.sync_copy(orow_v, out_out.at[pl.ds(base, BPW)])

        for q in range(NQ):
            cps = [
                pltpu.async_copy(
                    w_hbm.at[idx_v.at[c, pl.ds(q * QB, QB)]],
                    bufs.at[c], sem)
                for c in range(CTX)
            ]
            for cp in cps:
                cp.wait()

            def body(b, carry, q=q):
                for j in range(EMBED // 16):
                    a = bufs[0, b, pl.ds(j * 16, 16)]
                    for c in range(1, CTX):
                        a = a + bufs[c, b, pl.ds(j * 16, 16)]
                    acc_v[q * QB + b, pl.ds(j * 16, 16)] = a
                return carry

            lax.fori_loop(0, QB, body, 0)

        pltpu.sync_copy(acc_v, in_out.at[pl.ds(base, BPW)])

    return k(i_t, o, w)


def _tc_score(in_e, out_e):
    m_blk = 512

    def body(a_ref, b_ref, o_ref):
        a = a_ref[...].astype(jnp.bfloat16)
        bt = b_ref[...].astype(jnp.bfloat16)
        s = lax.dot_general(a, bt, (((1,), (1,)), ((), ())),
                            preferred_element_type=jnp.float32)
        o_ref[...] = jnp.minimum(s, 0.0) - jnp.log1p(jnp.exp(-jnp.abs(s)))

    return pl.pallas_call(
        body,
        grid=(B // m_blk,),
        in_specs=[
            pl.BlockSpec((m_blk, EMBED), lambda m: (m, 0)),
            pl.BlockSpec((B, EMBED), lambda m: (0, 0)),
        ],
        out_specs=pl.BlockSpec((m_blk, B), lambda m: (m, 0)),
        out_shape=jax.ShapeDtypeStruct((B, B), jnp.float32),
    )(in_e, out_e)


def kernel(i, o, W):
    i_t = i.T.astype(jnp.int32)
    in_e, out_e = _sc_gather(i_t, o.astype(jnp.int32), W)
    return _tc_score(in_e, out_e)


# speedup vs baseline: 1.2543x; 1.2543x over previous
import functools

import jax
import jax.numpy as jnp
from jax import lax
from jax.experimental import pallas as pl
from jax.experimental.pallas import tpu as pltpu
from jax.experimental.pallas import tpu_sc as plsc

B = 4096
CTX = 20
EMBED = 64
NC = 2
NS = 16
NW = NC * NS
BPW = B // NW
QB = 16
NQ = BPW // QB


def _sc_gather(rep_t, o_rep, w):
    mesh = plsc.VectorSubcoreMesh(core_axis_name="c", subcore_axis_name="s")

    @functools.partial(
        pl.kernel,
        mesh=mesh,
        out_type=(
            jax.ShapeDtypeStruct((B, EMBED), jnp.float32),
            jax.ShapeDtypeStruct((B, EMBED), jnp.float32),
        ),
        scratch_types=[
            pltpu.VMEM((CTX, 16 * QB), jnp.int32),
            pltpu.VMEM((16 * BPW,), jnp.int32),
            pltpu.VMEM((CTX * QB, EMBED), jnp.float32),
            pltpu.VMEM((BPW, EMBED), jnp.float32),
            pltpu.VMEM((BPW, EMBED), jnp.float32),
            pltpu.SemaphoreType.DMA,
            pltpu.SemaphoreType.DMA,
        ],
    )
    def k(rep_hbm, orep_hbm, w_hbm, in_out, out_out,
          rep_v, orep_v, bufs, acc_v, orow_v, osem, sem):
        wid = lax.axis_index("s") * NC + lax.axis_index("c")
        base = wid * BPW

        pltpu.sync_copy(orep_hbm.at[pl.ds(16 * base, 16 * BPW)], orep_v)

        def ofire(b, carry):
            s = orep_v[pl.ds(b * 16, 16)][0]
            pltpu.async_copy(w_hbm.at[pl.ds(s, 1), :],
                             orow_v.at[pl.ds(b, 1), :], osem)
            return carry

        lax.fori_loop(0, BPW, ofire, 0)
        pltpu.make_async_copy(w_hbm.at[pl.ds(0, BPW), :], orow_v, osem).wait()
        pltpu.sync_copy(orow_v, out_out.at[pl.ds(base, BPW)])

        for q in range(NQ):
            pltpu.sync_copy(
                rep_hbm.at[:, pl.ds(16 * (base + q * QB), 16 * QB)], rep_v)

            def fire(b, carry):
                for c in range(CTX):
                    s = rep_v[c, pl.ds(b * 16, 16)][0]
                    pltpu.async_copy(w_hbm.at[pl.ds(s, 1), :],
                                     bufs.at[pl.ds(c * QB + b, 1), :], sem)
                return carry

            lax.fori_loop(0, QB, fire, 0)
            pltpu.make_async_copy(w_hbm.at[pl.ds(0, CTX * QB), :], bufs,
                                  sem).wait()

            def body(b, carry, q=q):
                for j in range(EMBED // 16):
                    a = bufs[b, pl.ds(j * 16, 16)]
                    for c in range(1, CTX):
                        a = a + bufs[c * QB + b, pl.ds(j * 16, 16)]
                    acc_v[q * QB + b, pl.ds(j * 16, 16)] = a
                return carry

            lax.fori_loop(0, QB, body, 0)

        pltpu.sync_copy(acc_v, in_out.at[pl.ds(base, BPW)])

    return k(rep_t, o_rep, w)


def _tc_score(in_e, out_e):
    m_blk = 512

    def body(a_ref, b_ref, o_ref):
        a = a_ref[...].astype(jnp.bfloat16)
        bt = b_ref[...].astype(jnp.bfloat16)
        s = lax.dot_general(a, bt, (((1,), (1,)), ((), ())),
                            preferred_element_type=jnp.float32)
        o_ref[...] = jnp.minimum(s, 0.0) - jnp.log1p(jnp.exp(-jnp.abs(s)))

    return pl.pallas_call(
        body,
        grid=(B // m_blk,),
        in_specs=[
            pl.BlockSpec((m_blk, EMBED), lambda m: (m, 0)),
            pl.BlockSpec((B, EMBED), lambda m: (0, 0)),
        ],
        out_specs=pl.BlockSpec((m_blk, B), lambda m: (m, 0)),
        out_shape=jax.ShapeDtypeStruct((B, B), jnp.float32),
    )(in_e, out_e)


def kernel(i, o, W):
    i32 = i.astype(jnp.int32)
    o32 = o.astype(jnp.int32)
    rep_t = jnp.repeat(i32.T, 16, axis=1)
    o_rep = jnp.repeat(o32, 16)
    in_e, out_e = _sc_gather(rep_t, o_rep, W)
    return _tc_score(in_e, out_e)
